# baseline (speedup 1.0000x reference)
import functools

import jax
import jax.numpy as jnp
from jax import lax
from jax.experimental import pallas as pl
from jax.experimental.pallas import tpu as pltpu
from jax.experimental.pallas import tpu_sc as plsc

N_E = 1000000
N_R = 1000
K = 64
B = 16384

NC = 2
NS = 16
NW = NC * NS
BPW = B // NW
CH = 128
NCH = BPW // CH
NG = CH // 16


def _sc_distances(E2, R2, h2, t2, hp2, tp2, l2):
    mesh = plsc.VectorSubcoreMesh(
        core_axis_name="c", subcore_axis_name="s", num_cores=NC, num_subcores=NS)

    @functools.partial(
        pl.kernel,
        out_type=(
            jax.ShapeDtypeStruct((B // CH, CH), jnp.float32),
            jax.ShapeDtypeStruct((B // CH, CH), jnp.float32),
        ),
        mesh=mesh,
        compiler_params=pltpu.CompilerParams(
            needs_layout_passes=False, use_tc_tiling_on_sc=False),
        scratch_types=dict(
            ih=pltpu.VMEM((NCH, CH), jnp.int32),
            it=pltpu.VMEM((NCH, CH), jnp.int32),
            ihp=pltpu.VMEM((NCH, CH), jnp.int32),
            itp=pltpu.VMEM((NCH, CH), jnp.int32),
            il=pltpu.VMEM((NCH, CH), jnp.int32),
            la=pltpu.VMEM((2, NCH, CH), jnp.int32),
            lb=pltpu.VMEM((2, NCH, CH), jnp.int32),
            lr=pltpu.VMEM((NCH, CH), jnp.int32),
            ba0=pltpu.VMEM((CH, 2 * K), jnp.float32),
            bb0=pltpu.VMEM((CH, 2 * K), jnp.float32),
            br0=pltpu.VMEM((CH, 2 * K), jnp.float32),
            ba1=pltpu.VMEM((CH, 2 * K), jnp.float32),
            bb1=pltpu.VMEM((CH, 2 * K), jnp.float32),
            br1=pltpu.VMEM((CH, 2 * K), jnp.float32),
            d1v=pltpu.VMEM((NCH, CH), jnp.float32),
            d2v=pltpu.VMEM((NCH, CH), jnp.float32),
            sem0=pltpu.SemaphoreType.DMA,
            sem1=pltpu.SemaphoreType.DMA,
        ),
    )
    def k(E_h, R_h, h_h, t_h, hp_h, tp_h, l_h, d1_h, d2_h,
          ih, it, ihp, itp, il, la, lb, lr,
          ba0, bb0, br0, ba1, bb1, br1,
          d1v, d2v, sem0, sem1):
        wid = lax.axis_index("s") * NC + lax.axis_index("c")
        row0 = wid * NCH

        pltpu.sync_copy(h_h.at[pl.ds(row0, NCH)], ih)
        pltpu.sync_copy(t_h.at[pl.ds(row0, NCH)], it)
        pltpu.sync_copy(hp_h.at[pl.ds(row0, NCH)], ihp)
        pltpu.sync_copy(tp_h.at[pl.ds(row0, NCH)], itp)
        pltpu.sync_copy(l_h.at[pl.ds(row0, NCH)], il)

        def eline(v):
            return ((v >> _TSH) << (_TSH - 1)) | (v & (_TW // 2 - 1))

        def line_body(i, _):
            j = i // NG
            off = pl.multiple_of((i % NG) * 16, 16)
            la[0, j, pl.ds(off, 16)] = eline(ih[j, pl.ds(off, 16)])
            lb[0, j, pl.ds(off, 16)] = eline(it[j, pl.ds(off, 16)])
            la[1, j, pl.ds(off, 16)] = eline(ihp[j, pl.ds(off, 16)])
            lb[1, j, pl.ds(off, 16)] = eline(itp[j, pl.ds(off, 16)])
            lr[j, pl.ds(off, 16)] = il[j, pl.ds(off, 16)] >> 1
            return _

        lax.fori_loop(0, NCH * NG, line_body, None)

        iota16 = lax.iota(jnp.int32, 16)
        one16 = jnp.ones((16,), jnp.int32)

        bufs = [(ba0, bb0, br0, sem0), (ba1, bb1, br1, sem1)]

        def issue(p, j, bset):
            ba, bb, br, sem = bset
            return [
                pltpu.async_copy(E_h.at[la.at[p].at[j]], ba, sem),
                pltpu.async_copy(E_h.at[lb.at[p].at[j]], bb, sem),
                pltpu.async_copy(R_h.at[lr.at[j]], br, sem),
            ]

        def compute(p, j, iA, iB, dv, bset):
            ba, bb, br, _ = bset

            def group_body(g, _):
                off = pl.multiple_of(g * 16, 16)
                rows = off + iota16
                ca = ((iA[j, pl.ds(off, 16)] >> (_TSH - 1)) & one16) * K
                cb = ((iB[j, pl.ds(off, 16)] >> (_TSH - 1)) & one16) * K
                cl = (il[j, pl.ds(off, 16)] & one16) * K
                zero = jnp.zeros((16,), jnp.float32)
                acc = [zero, zero, zero, zero]
                for kk in range(K):
                    a = plsc.load_gather(ba, [rows, ca + kk])
                    b = plsc.load_gather(bb, [rows, cb + kk])
                    r = plsc.load_gather(br, [rows, cl + kk])
                    t1 = a + r - b
                    acc[kk % 4] = acc[kk % 4] + t1 * t1
                dv[j, pl.ds(off, 16)] = (acc[0] + acc[1]) + (acc[2] + acc[3])
                return _

            lax.fori_loop(0, NG, group_body, None)

        steps = [(0, j, ih, it, d1v) for j in range(NCH)] + [
            (1, j, ihp, itp, d2v) for j in range(NCH)]
        cps = issue(0, 0, bufs[0])
        for s, (p, j, iA, iB, dv) in enumerate(steps):
            if s + 1 < len(steps):
                pn, jn, _, _, _ = steps[s + 1]
                nxt = issue(pn, jn, bufs[(s + 1) % 2])
            else:
                nxt = None
            for cp in cps:
                cp.wait()
            compute(p, j, iA, iB, dv, bufs[s % 2])
            cps = nxt

        pltpu.sync_copy(d1v, d1_h.at[pl.ds(row0, NCH)])
        pltpu.sync_copy(d2v, d2_h.at[pl.ds(row0, NCH)])

    return k(E2, R2, h2, t2, hp2, tp2, l2)


_TW = 8192
_TSH = _TW.bit_length() - 1
_TGRID = -(-N_E // _TW)
_NLINES = _TGRID * (_TW // 2)


def _tc_retile(Et):
    def body(in_ref, out_ref):
        x = in_ref[...]
        ilo = jnp.eye(K, 2 * K, dtype=jnp.float32)
        ihi = jnp.eye(K, 2 * K, k=K, dtype=jnp.float32)
        lo = jnp.einsum("km,kn->mn", x[:, : _TW // 2], ilo,
                        preferred_element_type=jnp.float32)
        hi = jnp.einsum("km,kn->mn", x[:, _TW // 2 :], ihi,
                        preferred_element_type=jnp.float32)
        out_ref[...] = lo + hi

    return pl.pallas_call(
        body,
        grid=(_TGRID,),
        in_specs=[pl.BlockSpec((K, _TW), lambda i: (0, i))],
        out_specs=pl.BlockSpec((_TW // 2, 2 * K), lambda i: (i, 0)),
        out_shape=jax.ShapeDtypeStruct((_NLINES, 2 * K), jnp.float32),
        compiler_params=pltpu.CompilerParams(
            fuse_transposed_lhs_in_matmul=True),
    )(Et)


def _tc_finish(d1sq, d2sq):
    def fin(d1_ref, d2_ref, o_ref):
        d1 = jnp.sqrt(d1_ref[...])
        d2 = jnp.sqrt(d2_ref[...])
        v = jnp.maximum(d1 - d2 + 1.0, 0.0)
        o_ref[...] = jnp.sum(v, axis=(0, 1), keepdims=True) * (1.0 / B)

    return pl.pallas_call(
        fin,
        out_shape=jax.ShapeDtypeStruct((1, 1), jnp.float32),
    )(d1sq, d2sq)


def kernel(E, R, h, t, hp, tp, l):
    E2 = _tc_retile(E.T)
    R2 = R.reshape(N_R // 2, 2 * K)
    h2 = h.astype(jnp.int32).reshape(B // CH, CH)
    t2 = t.astype(jnp.int32).reshape(B // CH, CH)
    hp2 = hp.astype(jnp.int32).reshape(B // CH, CH)
    tp2 = tp.astype(jnp.int32).reshape(B // CH, CH)
    l2 = l.astype(jnp.int32).reshape(B // CH, CH)
    d1sq, d2sq = _sc_distances(E2, R2, h2, t2, hp2, tp2, l2)
    out = _tc_finish(d1sq, d2sq)
    return out[0, 0]


# speedup vs baseline: 1.0136x; 1.0136x over previous
import functools

import jax
import jax.numpy as jnp
from jax import lax
from jax.experimental import pallas as pl
from jax.experimental.pallas import tpu as pltpu
from jax.experimental.pallas import tpu_sc as plsc

N_E = 1000000
N_R = 1000
K = 64
B = 16384

NC = 2
NS = 16
NW = NC * NS
BPW = B // NW
CH = 128
NCH = BPW // CH
NG = CH // 16


def _sc_distances(E3, Rc, h2, t2, hp2, tp2, l2):
    mesh = plsc.VectorSubcoreMesh(
        core_axis_name="c", subcore_axis_name="s", num_cores=NC, num_subcores=NS)

    @functools.partial(
        pl.kernel,
        out_type=(
            jax.ShapeDtypeStruct((B // CH, CH), jnp.float32),
            jax.ShapeDtypeStruct((B // CH, CH), jnp.float32),
        ),
        mesh=mesh,
        compiler_params=pltpu.CompilerParams(
            needs_layout_passes=False, use_tc_tiling_on_sc=False),
        scratch_types=dict(
            ih=pltpu.VMEM((NCH, CH), jnp.int32),
            it=pltpu.VMEM((NCH, CH), jnp.int32),
            ihp=pltpu.VMEM((NCH, CH), jnp.int32),
            itp=pltpu.VMEM((NCH, CH), jnp.int32),
            il=pltpu.VMEM((NCH, CH), jnp.int32),
            ba0=pltpu.VMEM((CH, K), jnp.float32),
            bb0=pltpu.VMEM((CH, K), jnp.float32),
            bc0=pltpu.VMEM((CH, K), jnp.float32),
            bd0=pltpu.VMEM((CH, K), jnp.float32),
            br0=pltpu.VMEM((CH, K), jnp.float32),
            ba1=pltpu.VMEM((CH, K), jnp.float32),
            bb1=pltpu.VMEM((CH, K), jnp.float32),
            bc1=pltpu.VMEM((CH, K), jnp.float32),
            bd1=pltpu.VMEM((CH, K), jnp.float32),
            br1=pltpu.VMEM((CH, K), jnp.float32),
            d1v=pltpu.VMEM((NCH, CH), jnp.float32),
            d2v=pltpu.VMEM((NCH, CH), jnp.float32),
            sem0=pltpu.SemaphoreType.DMA,
            sem1=pltpu.SemaphoreType.DMA,
        ),
    )
    def k(E_h, R_h, h_h, t_h, hp_h, tp_h, l_h, d1_h, d2_h,
          ih, it, ihp, itp, il,
          ba0, bb0, bc0, bd0, br0, ba1, bb1, bc1, bd1, br1,
          d1v, d2v, sem0, sem1):
        wid = lax.axis_index("s") * NC + lax.axis_index("c")
        row0 = wid * NCH

        pltpu.sync_copy(h_h.at[pl.ds(row0, NCH)], ih)
        pltpu.sync_copy(t_h.at[pl.ds(row0, NCH)], it)
        pltpu.sync_copy(hp_h.at[pl.ds(row0, NCH)], ihp)
        pltpu.sync_copy(tp_h.at[pl.ds(row0, NCH)], itp)
        pltpu.sync_copy(l_h.at[pl.ds(row0, NCH)], il)

        def erow(v):
            return (((v >> _TSH) << _TSH)
                    | ((v & (_TW // 2 - 1)) << 1)
                    | ((v >> (_TSH - 1)) & 1))

        def line_body(i, _):
            j = i // NG
            off = pl.multiple_of((i % NG) * 16, 16)
            ih[j, pl.ds(off, 16)] = erow(ih[j, pl.ds(off, 16)])
            it[j, pl.ds(off, 16)] = erow(it[j, pl.ds(off, 16)])
            ihp[j, pl.ds(off, 16)] = erow(ihp[j, pl.ds(off, 16)])
            itp[j, pl.ds(off, 16)] = erow(itp[j, pl.ds(off, 16)])
            return _

        lax.fori_loop(0, NCH * NG, line_body, None)

        iota16 = lax.iota(jnp.int32, 16)

        bufs = [(ba0, bb0, bc0, bd0, br0, sem0),
                (ba1, bb1, bc1, bd1, br1, sem1)]

        def issue(j, bset):
            ba, bb, bc, bd, br, sem = bset
            return [
                pltpu.async_copy(E_h.at[ih.at[j]], ba, sem),
                pltpu.async_copy(E_h.at[it.at[j]], bb, sem),
                pltpu.async_copy(E_h.at[ihp.at[j]], bc, sem),
                pltpu.async_copy(E_h.at[itp.at[j]], bd, sem),
                pltpu.async_copy(R_h.at[il.at[j]], br, sem),
            ]

        def compute(j, bset):
            ba, bb, bc, bd, br, _ = bset

            def group_body(g, _):
                off = pl.multiple_of(g * 16, 16)
                rows = off + iota16
                zero = jnp.zeros((16,), jnp.float32)
                cz = il[j, pl.ds(off, 16)] >> 20
                a1 = [zero, zero, zero, zero]
                a2 = [zero, zero, zero, zero]
                for kk in range(K):
                    col = cz + kk
                    a = plsc.load_gather(ba, [rows, col])
                    b = plsc.load_gather(bb, [rows, col])
                    c = plsc.load_gather(bc, [rows, col])
                    d = plsc.load_gather(bd, [rows, col])
                    r = plsc.load_gather(br, [rows, col])
                    t1 = a + r - b
                    t2 = c + r - d
                    a1[kk % 4] = a1[kk % 4] + t1 * t1
                    a2[kk % 4] = a2[kk % 4] + t2 * t2
                d1v[j, pl.ds(off, 16)] = (a1[0] + a1[1]) + (a1[2] + a1[3])
                d2v[j, pl.ds(off, 16)] = (a2[0] + a2[1]) + (a2[2] + a2[3])
                return _

            lax.fori_loop(0, NG, group_body, None)

        cps = issue(0, bufs[0])
        for j in range(NCH):
            nxt = issue(j + 1, bufs[(j + 1) % 2]) if j + 1 < NCH else None
            for cp in cps:
                cp.wait()
            compute(j, bufs[j % 2])
            cps = nxt

        pltpu.sync_copy(d1v, d1_h.at[pl.ds(row0, NCH)])
        pltpu.sync_copy(d2v, d2_h.at[pl.ds(row0, NCH)])

    return k(E3, Rc, h2, t2, hp2, tp2, l2)


_TW = 8192
_TSH = _TW.bit_length() - 1
_TGRID = -(-N_E // _TW)
_NLINES = _TGRID * (_TW // 2)


def _tc_retile(Et):
    def body(in_ref, out_ref):
        x = in_ref[...]
        ilo = jnp.eye(K, 2 * K, dtype=jnp.float32)
        ihi = jnp.eye(K, 2 * K, k=K, dtype=jnp.float32)
        lo = jnp.einsum("km,kn->mn", x[:, : _TW // 2], ilo,
                        preferred_element_type=jnp.float32)
        hi = jnp.einsum("km,kn->mn", x[:, _TW // 2 :], ihi,
                        preferred_element_type=jnp.float32)
        out_ref[...] = lo + hi

    return pl.pallas_call(
        body,
        grid=(_TGRID,),
        in_specs=[pl.BlockSpec((K, _TW), lambda i: (0, i))],
        out_specs=pl.BlockSpec((_TW // 2, 2 * K), lambda i: (i, 0)),
        out_shape=jax.ShapeDtypeStruct((_NLINES, 2 * K), jnp.float32),
        compiler_params=pltpu.CompilerParams(
            fuse_transposed_lhs_in_matmul=True),
    )(Et)


def _tc_finish(d1sq, d2sq):
    def fin(d1_ref, d2_ref, o_ref):
        d1 = jnp.sqrt(d1_ref[...])
        d2 = jnp.sqrt(d2_ref[...])
        v = jnp.maximum(d1 - d2 + 1.0, 0.0)
        o_ref[...] = jnp.sum(v, axis=(0, 1), keepdims=True) * (1.0 / B)

    return pl.pallas_call(
        fin,
        out_shape=jax.ShapeDtypeStruct((1, 1), jnp.float32),
    )(d1sq, d2sq)


def kernel(E, R, h, t, hp, tp, l):
    E3 = _tc_retile(E.T).reshape(2 * _NLINES, K)
    h2 = h.astype(jnp.int32).reshape(B // CH, CH)
    t2 = t.astype(jnp.int32).reshape(B // CH, CH)
    hp2 = hp.astype(jnp.int32).reshape(B // CH, CH)
    tp2 = tp.astype(jnp.int32).reshape(B // CH, CH)
    l2 = l.astype(jnp.int32).reshape(B // CH, CH)
    d1sq, d2sq = _sc_distances(E3, R, h2, t2, hp2, tp2, l2)
    out = _tc_finish(d1sq, d2sq)
    return out[0, 0]


# speedup vs baseline: 1.2793x; 1.2621x over previous
import functools

import jax
import jax.numpy as jnp
from jax import lax
from jax.experimental import pallas as pl
from jax.experimental.pallas import tpu as pltpu
from jax.experimental.pallas import tpu_sc as plsc

N_E = 1000000
N_R = 1000
K = 64
B = 16384

NC = 2
NS = 16
NW = NC * NS
BPW = B // NW
CH = 128
NCH = BPW // CH
NG = CH // 16


def _sc_distances(E3, Rc, h2, t2, hp2, tp2, l2):
    mesh = plsc.VectorSubcoreMesh(
        core_axis_name="c", subcore_axis_name="s", num_cores=NC, num_subcores=NS)

    @functools.partial(
        pl.kernel,
        out_type=(
            jax.ShapeDtypeStruct((B // CH, CH), jnp.float32),
            jax.ShapeDtypeStruct((B // CH, CH), jnp.float32),
        ),
        mesh=mesh,
        compiler_params=pltpu.CompilerParams(
            needs_layout_passes=False, use_tc_tiling_on_sc=False),
        scratch_types=dict(
            ih=pltpu.VMEM((NCH, CH), jnp.int32),
            it=pltpu.VMEM((NCH, CH), jnp.int32),
            ihp=pltpu.VMEM((NCH, CH), jnp.int32),
            itp=pltpu.VMEM((NCH, CH), jnp.int32),
            il=pltpu.VMEM((NCH, CH), jnp.int32),
            ba0=pltpu.VMEM((CH, K), jnp.float32),
            bb0=pltpu.VMEM((CH, K), jnp.float32),
            bc0=pltpu.VMEM((CH, K), jnp.float32),
            bd0=pltpu.VMEM((CH, K), jnp.float32),
            br0=pltpu.VMEM((CH, K), jnp.float32),
            ba1=pltpu.VMEM((CH, K), jnp.float32),
            bb1=pltpu.VMEM((CH, K), jnp.float32),
            bc1=pltpu.VMEM((CH, K), jnp.float32),
            bd1=pltpu.VMEM((CH, K), jnp.float32),
            br1=pltpu.VMEM((CH, K), jnp.float32),
            d1v=pltpu.VMEM((NCH, CH), jnp.float32),
            d2v=pltpu.VMEM((NCH, CH), jnp.float32),
            sem0=pltpu.SemaphoreType.DMA,
            sem1=pltpu.SemaphoreType.DMA,
        ),
    )
    def k(E_h, R_h, h_h, t_h, hp_h, tp_h, l_h, d1_h, d2_h,
          ih, it, ihp, itp, il,
          ba0, bb0, bc0, bd0, br0, ba1, bb1, bc1, bd1, br1,
          d1v, d2v, sem0, sem1):
        wid = lax.axis_index("s") * NC + lax.axis_index("c")
        row0 = wid * NCH

        pltpu.sync_copy(h_h.at[pl.ds(row0, NCH)], ih)
        pltpu.sync_copy(t_h.at[pl.ds(row0, NCH)], it)
        pltpu.sync_copy(hp_h.at[pl.ds(row0, NCH)], ihp)
        pltpu.sync_copy(tp_h.at[pl.ds(row0, NCH)], itp)
        pltpu.sync_copy(l_h.at[pl.ds(row0, NCH)], il)

        def erow(v):
            return (((v >> _TSH) << _TSH)
                    | ((v & (_TW // 2 - 1)) << 1)
                    | ((v >> (_TSH - 1)) & 1))

        def line_body(i, _):
            j = i // NG
            off = pl.multiple_of((i % NG) * 16, 16)
            ih[j, pl.ds(off, 16)] = erow(ih[j, pl.ds(off, 16)])
            it[j, pl.ds(off, 16)] = erow(it[j, pl.ds(off, 16)])
            ihp[j, pl.ds(off, 16)] = erow(ihp[j, pl.ds(off, 16)])
            itp[j, pl.ds(off, 16)] = erow(itp[j, pl.ds(off, 16)])
            return _

        lax.fori_loop(0, NCH * NG, line_body, None)

        iota16 = lax.iota(jnp.int32, 16)

        bufs = [(ba0, bb0, bc0, bd0, br0, sem0),
                (ba1, bb1, bc1, bd1, br1, sem1)]

        def issue(j, bset):
            ba, bb, bc, bd, br, sem = bset
            return [
                pltpu.async_copy(E_h.at[ih.at[j]], ba, sem),
                pltpu.async_copy(E_h.at[it.at[j]], bb, sem),
                pltpu.async_copy(E_h.at[ihp.at[j]], bc, sem),
                pltpu.async_copy(E_h.at[itp.at[j]], bd, sem),
                pltpu.async_copy(R_h.at[il.at[j]], br, sem),
            ]

        def compute(j, bset):
            ba, bb, bc, bd, br, _ = bset

            def group_body(g, _):
                off = pl.multiple_of(g * 16, 16)
                rows = off + iota16
                zero = jnp.zeros((16,), jnp.float32)
                cz = (il[j, pl.ds(off, 16)] >> 20) + iota16
                a1 = [zero, zero, zero, zero]
                a2 = [zero, zero, zero, zero]
                for kk in range(K):
                    col = (cz + kk) & (K - 1)
                    a = plsc.load_gather(ba, [rows, col])
                    b = plsc.load_gather(bb, [rows, col])
                    c = plsc.load_gather(bc, [rows, col])
                    d = plsc.load_gather(bd, [rows, col])
                    r = plsc.load_gather(br, [rows, col])
                    t1 = a + r - b
                    t2 = c + r - d
                    a1[kk % 4] = a1[kk % 4] + t1 * t1
                    a2[kk % 4] = a2[kk % 4] + t2 * t2
                d1v[j, pl.ds(off, 16)] = (a1[0] + a1[1]) + (a1[2] + a1[3])
                d2v[j, pl.ds(off, 16)] = (a2[0] + a2[1]) + (a2[2] + a2[3])
                return _

            lax.fori_loop(0, NG, group_body, None)

        cps = issue(0, bufs[0])
        for j in range(NCH):
            nxt = issue(j + 1, bufs[(j + 1) % 2]) if j + 1 < NCH else None
            for cp in cps:
                cp.wait()
            compute(j, bufs[j % 2])
            cps = nxt

        pltpu.sync_copy(d1v, d1_h.at[pl.ds(row0, NCH)])
        pltpu.sync_copy(d2v, d2_h.at[pl.ds(row0, NCH)])

    return k(E3, Rc, h2, t2, hp2, tp2, l2)


_TW = 8192
_TSH = _TW.bit_length() - 1
_TGRID = -(-N_E // _TW)
_NLINES = _TGRID * (_TW // 2)


def _tc_retile(Et):
    def body(in_ref, out_ref):
        x = in_ref[...]
        ilo = jnp.eye(K, 2 * K, dtype=jnp.float32)
        ihi = jnp.eye(K, 2 * K, k=K, dtype=jnp.float32)
        lo = jnp.einsum("km,kn->mn", x[:, : _TW // 2], ilo,
                        preferred_element_type=jnp.float32)
        hi = jnp.einsum("km,kn->mn", x[:, _TW // 2 :], ihi,
                        preferred_element_type=jnp.float32)
        out_ref[...] = lo + hi

    return pl.pallas_call(
        body,
        grid=(_TGRID,),
        in_specs=[pl.BlockSpec((K, _TW), lambda i: (0, i))],
        out_specs=pl.BlockSpec((_TW // 2, 2 * K), lambda i: (i, 0)),
        out_shape=jax.ShapeDtypeStruct((_NLINES, 2 * K), jnp.float32),
        compiler_params=pltpu.CompilerParams(
            fuse_transposed_lhs_in_matmul=True),
    )(Et)


def _tc_finish(d1sq, d2sq):
    def fin(d1_ref, d2_ref, o_ref):
        d1 = jnp.sqrt(d1_ref[...])
        d2 = jnp.sqrt(d2_ref[...])
        v = jnp.maximum(d1 - d2 + 1.0, 0.0)
        o_ref[...] = jnp.sum(v, axis=(0, 1), keepdims=True) * (1.0 / B)

    return pl.pallas_call(
        fin,
        out_shape=jax.ShapeDtypeStruct((1, 1), jnp.float32),
    )(d1sq, d2sq)


def kernel(E, R, h, t, hp, tp, l):
    E3 = _tc_retile(E.T).reshape(2 * _NLINES, K)
    h2 = h.astype(jnp.int32).reshape(B // CH, CH)
    t2 = t.astype(jnp.int32).reshape(B // CH, CH)
    hp2 = hp.astype(jnp.int32).reshape(B // CH, CH)
    tp2 = tp.astype(jnp.int32).reshape(B // CH, CH)
    l2 = l.astype(jnp.int32).reshape(B // CH, CH)
    d1sq, d2sq = _sc_distances(E3, R, h2, t2, hp2, tp2, l2)
    out = _tc_finish(d1sq, d2sq)
    return out[0, 0]
